# Initial kernel scaffold; baseline (speedup 1.0000x reference)
import jax
import jax.numpy as jnp
from jax.experimental import pallas as pl


def kernel(logits):
    raise NotImplementedError("write your pallas kernel here")


# speedup vs baseline: 2.9626x; 2.9626x over previous
import functools

import numpy as np
import jax
import jax.numpy as jnp
from jax.experimental import pallas as pl
from jax.experimental.pallas import tpu as pltpu

_K = 32


def _bits_key42(idx_u32):
    ks = (np.uint32(0), np.uint32(42), np.uint32(42 ^ 0x1BD11BDA))
    rot = ((13, 15, 26, 6), (17, 29, 16, 24))
    x0 = jnp.zeros_like(idx_u32)
    x1 = idx_u32 + ks[1]
    for i in range(5):
        for r in rot[i % 2]:
            x0 = x0 + x1
            x1 = (x1 << np.uint32(r)) | (x1 >> np.uint32(32 - r))
            x1 = x1 ^ x0
        x0 = x0 + ks[(i + 1) % 3]
        x1 = x1 + ks[(i + 2) % 3] + np.uint32(i + 1)
    return x0 ^ x1


def _block_body(l_ref, o_ref, *, bm, n):
    i = pl.program_id(0)
    l = l_ref[...]

    r = jax.lax.broadcasted_iota(jnp.int32, (bm, n), 0)
    c = jax.lax.broadcasted_iota(jnp.int32, (bm, n), 1)
    flat = (i * bm + r) * n + c
    bits = _bits_key42(flat.astype(jnp.uint32))
    u = jax.lax.bitcast_convert_type(
        (bits >> np.uint32(9)) | np.uint32(0x3F800000), jnp.float32) - 1.0
    u = jnp.maximum(u, 0.0)
    g = -jnp.log(-jnp.log(u + 1e-8) + 1e-8)
    pert = l + g

    b = jax.lax.bitcast_convert_type(pert, jnp.uint32)
    su = b ^ (np.uint32(0x80000000) | (np.uint32(0) - (b >> np.uint32(31))))
    s2 = jax.lax.bitcast_convert_type(su ^ np.uint32(0x80000000), jnp.int32)

    p = jnp.zeros((bm, 1), jnp.uint32)
    for bit in range(31, -1, -1):
        cand = p | np.uint32(1 << bit)
        cand2 = jax.lax.bitcast_convert_type(
            cand ^ np.uint32(0x80000000), jnp.int32)
        cnt = jnp.sum((s2 >= cand2).astype(jnp.int32), axis=1, keepdims=True)
        p = jnp.where(cnt >= _K, cand, p)
    thr2 = jax.lax.bitcast_convert_type(p ^ np.uint32(0x80000000), jnp.int32)

    mask = s2 >= thr2
    lm = jnp.where(mask, l, -jnp.inf)
    m = jnp.max(lm, axis=1, keepdims=True)
    e = jnp.where(mask, jnp.exp(l - m), 0.0)
    d = jnp.sum(e, axis=1, keepdims=True)
    o_ref[...] = e / d


@jax.jit
def kernel(logits):
    rows, n = logits.shape
    bm = 256 if rows % 256 == 0 else 8
    grid = (rows // bm,)
    return pl.pallas_call(
        functools.partial(_block_body, bm=bm, n=n),
        grid=grid,
        in_specs=[pl.BlockSpec((bm, n), lambda i: (i, 0))],
        out_specs=pl.BlockSpec((bm, n), lambda i: (i, 0)),
        out_shape=jax.ShapeDtypeStruct((rows, n), jnp.float32),
        compiler_params=pltpu.CompilerParams(
            dimension_semantics=("arbitrary",)),
    )(logits)


# speedup vs baseline: 3.0084x; 1.0155x over previous
import functools

import numpy as np
import jax
import jax.numpy as jnp
from jax.experimental import pallas as pl
from jax.experimental.pallas import tpu as pltpu

_K = 32


def _bits_key42(idx_u32):
    ks = (np.uint32(0), np.uint32(42), np.uint32(42 ^ 0x1BD11BDA))
    rot = ((13, 15, 26, 6), (17, 29, 16, 24))
    x0 = jnp.zeros_like(idx_u32)
    x1 = idx_u32 + ks[1]
    for i in range(5):
        for r in rot[i % 2]:
            x0 = x0 + x1
            x1 = (x1 << np.uint32(r)) | (x1 >> np.uint32(32 - r))
            x1 = x1 ^ x0
        x0 = x0 + ks[(i + 1) % 3]
        x1 = x1 + ks[(i + 2) % 3] + np.uint32(i + 1)
    return x0 ^ x1


def _block_body(l_ref, o_ref, *, bm, n):
    i = pl.program_id(0)
    l = l_ref[...]

    r = jax.lax.broadcasted_iota(jnp.int32, (bm, n), 0)
    c = jax.lax.broadcasted_iota(jnp.int32, (bm, n), 1)
    flat = (i * bm + r) * n + c
    bits = _bits_key42(flat.astype(jnp.uint32))
    u = jax.lax.bitcast_convert_type(
        (bits >> np.uint32(9)) | np.uint32(0x3F800000), jnp.float32) - 1.0
    g = -jnp.log(-jnp.log(u + 1e-8) + 1e-8)
    pert = l + g

    b = jax.lax.bitcast_convert_type(pert, jnp.uint32)
    su = b ^ (np.uint32(0x80000000) | (np.uint32(0) - (b >> np.uint32(31))))
    s2 = jax.lax.bitcast_convert_type(su ^ np.uint32(0x80000000), jnp.int32)

    p = jnp.zeros((bm, 1), jnp.uint32)
    for bit in range(31, -1, -1):
        cand = p | np.uint32(1 << bit)
        cand2 = jax.lax.bitcast_convert_type(
            cand ^ np.uint32(0x80000000), jnp.int32)
        cnt = jnp.sum((s2 >= cand2).astype(jnp.int32), axis=1, keepdims=True)
        p = jnp.where(cnt >= _K, cand, p)
    thr2 = jax.lax.bitcast_convert_type(p ^ np.uint32(0x80000000), jnp.int32)

    mask = s2 >= thr2
    lm = jnp.where(mask, l, -jnp.inf)
    m = jnp.max(lm, axis=1, keepdims=True)
    e = jnp.exp(lm - m)
    d = jnp.sum(e, axis=1, keepdims=True)
    o_ref[...] = e * (1.0 / d)


@jax.jit
def kernel(logits):
    rows, n = logits.shape
    bm = 256 if rows % 256 == 0 else 8
    grid = (rows // bm,)
    return pl.pallas_call(
        functools.partial(_block_body, bm=bm, n=n),
        grid=grid,
        in_specs=[pl.BlockSpec((bm, n), lambda i: (i, 0))],
        out_specs=pl.BlockSpec((bm, n), lambda i: (i, 0)),
        out_shape=jax.ShapeDtypeStruct((rows, n), jnp.float32),
        compiler_params=pltpu.CompilerParams(
            dimension_semantics=("arbitrary",)),
    )(logits)
